# baseline (speedup 1.0000x reference)
import functools

import jax
import jax.numpy as jnp
from jax import lax
from jax.experimental import pallas as pl
from jax.experimental.pallas import tpu as pltpu
from jax.experimental.pallas import tpu_sc as plsc

_DIM = 64
_LANES = 2 * _DIM
_PACK = 2 * _LANES


def _sc_worker_count():
    try:
        info = plsc.get_sparse_core_info()
        return info.num_cores, info.num_subcores
    except Exception:
        return 2, 16


def _make_sc_gather(batch, pack):
    nc, ns = _sc_worker_count()
    nw = nc * ns
    b_per_w = batch // nw
    mesh = plsc.VectorSubcoreMesh(core_axis_name="c", subcore_axis_name="s")

    @functools.partial(
        pl.kernel,
        mesh=mesh,
        out_type=jax.ShapeDtypeStruct((batch, pack), jnp.float32),
        scratch_types=[
            pltpu.VMEM((b_per_w,), jnp.int32),
            pltpu.VMEM((b_per_w, pack), jnp.float32),
            pltpu.SemaphoreType.DMA,
        ],
    )
    def sc_gather(table_hbm, idx_hbm, out_hbm, idx_v, rows_v, sem):
        wid = lax.axis_index("s") * nc + lax.axis_index("c")
        base = wid * b_per_w
        pltpu.sync_copy(idx_hbm.at[pl.ds(base, b_per_w)], idx_v)
        pltpu.async_copy(table_hbm.at[idx_v], rows_v, sem).wait()
        pltpu.sync_copy(rows_v, out_hbm.at[pl.ds(base, b_per_w)])

    return sc_gather


def _apply_body(x_ref, gb_ref, o_ref):
    g = gb_ref[:, 0:_DIM]
    b = gb_ref[:, _DIM : 2 * _DIM]
    o_ref[...] = x_ref[...] * g[:, None, :] + b[:, None, :]


def kernel(x, subject_id, gamma_w, beta_w):
    batch, seq, dim = x.shape
    idx = subject_id.astype(jnp.int32)
    table = jnp.concatenate([gamma_w, beta_w], axis=1)

    gb = _make_sc_gather(batch, 2 * dim)(table, idx)

    b_blk = 128
    out = pl.pallas_call(
        _apply_body,
        grid=(batch // b_blk,),
        in_specs=[
            pl.BlockSpec((b_blk, seq, dim), lambda i: (i, 0, 0)),
            pl.BlockSpec((b_blk, 2 * dim), lambda i: (i, 0)),
        ],
        out_specs=pl.BlockSpec((b_blk, seq, dim), lambda i: (i, 0, 0)),
        out_shape=jax.ShapeDtypeStruct((batch, seq, dim), jnp.float32),
    )(x, gb)
    return out


# speedup vs baseline: 1.6219x; 1.6219x over previous
import functools

import jax
import jax.numpy as jnp
from jax import lax
from jax.experimental import pallas as pl
from jax.experimental.pallas import tpu as pltpu
from jax.experimental.pallas import tpu_sc as plsc

_DIM = 64
_LANES = 2 * _DIM
_PACK = 2 * _LANES


def _sc_worker_count():
    try:
        info = plsc.get_sparse_core_info()
        return info.num_cores, info.num_subcores
    except Exception:
        return 2, 16


def _make_sc_gather(batch, pack):
    nc, ns = _sc_worker_count()
    nw = nc * ns
    b_per_w = batch // nw
    mesh = plsc.VectorSubcoreMesh(core_axis_name="c", subcore_axis_name="s")

    @functools.partial(
        pl.kernel,
        mesh=mesh,
        out_type=jax.ShapeDtypeStruct((batch, pack), jnp.float32),
        scratch_types=[
            pltpu.VMEM((b_per_w,), jnp.int32),
            pltpu.VMEM((b_per_w, pack), jnp.float32),
            pltpu.SemaphoreType.DMA,
        ],
    )
    def sc_gather(table_hbm, idx_hbm, out_hbm, idx_v, rows_v, sem):
        wid = lax.axis_index("s") * nc + lax.axis_index("c")
        base = wid * b_per_w
        pltpu.sync_copy(idx_hbm.at[pl.ds(base, b_per_w)], idx_v)
        pltpu.async_copy(table_hbm.at[idx_v], rows_v, sem).wait()
        pltpu.sync_copy(rows_v, out_hbm.at[pl.ds(base, b_per_w)])

    return sc_gather


def _apply_body(x_ref, gb_ref, o_ref):
    g = gb_ref[:, 0:_LANES]
    b = gb_ref[:, _LANES:_PACK]
    n_strips = x_ref.shape[1] // _LANES
    for c in range(n_strips):
        sl = slice(c * _LANES, (c + 1) * _LANES)
        o_ref[:, sl] = x_ref[:, sl] * g + b


def kernel(x, subject_id, gamma_w, beta_w):
    batch, seq, dim = x.shape
    idx = subject_id.astype(jnp.int32)
    table = jnp.concatenate([gamma_w, gamma_w, beta_w, beta_w], axis=1)

    gb = _make_sc_gather(batch, _PACK)(table, idx)

    row = seq * dim
    x2 = x.reshape(batch, row)
    b_blk = 64
    out2 = pl.pallas_call(
        _apply_body,
        grid=(batch // b_blk,),
        in_specs=[
            pl.BlockSpec((b_blk, row), lambda i: (i, 0)),
            pl.BlockSpec((b_blk, _PACK), lambda i: (i, 0)),
        ],
        out_specs=pl.BlockSpec((b_blk, row), lambda i: (i, 0)),
        out_shape=jax.ShapeDtypeStruct((batch, row), jnp.float32),
    )(x2, gb)
    return out2.reshape(batch, seq, dim)
